# Initial kernel scaffold; baseline (speedup 1.0000x reference)
import jax
import jax.numpy as jnp
from jax.experimental import pallas as pl


def kernel(pos, z, batch, edge_index, W1_0, W2_0, W1_1, W2_1, W_last, b_last):
    raise NotImplementedError("write your pallas kernel here")


# speedup vs baseline: 9.9988x; 9.9988x over previous
import functools

import jax
import jax.numpy as jnp
from jax import lax
from jax.experimental import pallas as pl
from jax.experimental.pallas import tpu as pltpu
from jax.experimental.pallas import tpu_sc as plsc

N = 100000
E = 3200000
B = 2000
D = 8

CHUNK = 128
NCH = E // CHUNK
NC, NS = 2, 16
NW = NC * NS
CPW = -(-NCH // NW)
NPAD = 782 * 128
NCHN = NPAD // CHUNK
AGGW = NPAD * D
BPAD = 2048

_mesh = plsc.VectorSubcoreMesh(core_axis_name="c", subcore_axis_name="s")


def _edge_hist_body(src_hbm, dst_hbm, pos_hbm, z_hbm, zeros_hbm,
                    w_hbm, agg_hbm,
                    idx_s, idx_d, prow_s, prow_d, wbuf, flatbuf, zbuf,
                    aggsp, sem):
    c = lax.axis_index("c")
    s = lax.axis_index("s")
    wid = s * NC + c
    seg = AGGW // NS
    pltpu.sync_copy(zeros_hbm.at[pl.ds(s * seg, seg)],
                    aggsp.at[pl.ds(s * seg, seg)])
    plsc.subcore_barrier()

    iota = lax.iota(jnp.int32, 16)

    def body(i, carry):
        g = wid + NW * i

        @pl.when(g < NCH)
        def _():
            pltpu.sync_copy(src_hbm.at[g], idx_s)
            pltpu.sync_copy(dst_hbm.at[g], idx_d)
            pltpu.async_copy(pos_hbm.at[idx_s], prow_s, sem).wait()
            pltpu.async_copy(pos_hbm.at[idx_d], prow_d, sem).wait()
            pltpu.async_copy(z_hbm.at[idx_s], zbuf, sem).wait()
            for t in range(CHUNK // 16):
                dst16 = idx_d[pl.ds(t * 16, 16)]
                e16 = t * 16 + iota
                acc = jnp.full((16,), 1e-12, jnp.float32)
                for j in range(3):
                    js = jnp.full((16,), j, jnp.int32)
                    ps = plsc.load_gather(prow_s, [e16, js])
                    pd = plsc.load_gather(prow_d, [e16, js])
                    dd = pd - ps
                    acc = acc + dd * dd
                ib = plsc.bitcast(acc, jnp.int32)
                r = plsc.bitcast(jnp.int32(0x5F3759DF) - (ib >> 1), jnp.float32)
                for _ in range(3):
                    r = r * (1.5 - 0.5 * acc * r * r)
                w16 = jnp.exp(-(acc * r))
                zs = zbuf[pl.ds(t * 16, 16)]
                wbuf[pl.ds(t * 16, 16)] = w16
                flatbuf[pl.ds(t * 16, 16)] = dst16 * D + zs
            pltpu.sync_copy(wbuf, w_hbm.at[g])
            pltpu.sync_copy(wbuf, aggsp.at[flatbuf], add=True)

        return carry

    lax.fori_loop(0, CPW, body, 0)
    plsc.subcore_barrier()
    pltpu.sync_copy(aggsp.at[pl.ds(s * seg, seg)],
                    agg_hbm.at[c, pl.ds(s * seg, seg)])


_edge_hist = pl.kernel(
    _edge_hist_body,
    out_type=(jax.ShapeDtypeStruct((NCH, CHUNK), jnp.float32),
              jax.ShapeDtypeStruct((NC, AGGW), jnp.float32)),
    mesh=_mesh,
    compiler_params=pltpu.CompilerParams(needs_layout_passes=False, use_tc_tiling_on_sc=False),
    scratch_types=[
        pltpu.VMEM((CHUNK,), jnp.int32),
        pltpu.VMEM((CHUNK,), jnp.int32),
        pltpu.VMEM((CHUNK, 8), jnp.float32),
        pltpu.VMEM((CHUNK, 8), jnp.float32),
        pltpu.VMEM((CHUNK,), jnp.float32),
        pltpu.VMEM((CHUNK,), jnp.int32),
        pltpu.VMEM((CHUNK,), jnp.int32),
        pltpu.VMEM_SHARED((AGGW,), jnp.float32),
        pltpu.SemaphoreType.DMA,
    ],
)


def _spmv_body(src_hbm, dst_hbm, w_hbm, h_hbm, zeros_hbm, agg_hbm,
               idx_s, idx_d, wv, rows, aggsp, sem):
    c = lax.axis_index("c")
    s = lax.axis_index("s")
    wid = s * NC + c
    seg = NPAD // NS
    pltpu.sync_copy(zeros_hbm.at[pl.ds(s * seg, seg)],
                    aggsp.at[pl.ds(s * seg, seg)])
    plsc.subcore_barrier()

    iota = lax.iota(jnp.int32, 16)
    colidx = iota & (D - 1)

    def body(i, carry):
        g = wid + NW * i

        @pl.when(g < NCH)
        def _():
            pltpu.sync_copy(src_hbm.at[g], idx_s)
            pltpu.sync_copy(dst_hbm.at[g], idx_d)
            pltpu.sync_copy(w_hbm.at[g], wv)
            pltpu.async_copy(h_hbm.at[idx_s], rows, sem).wait()

            def mul_body(v, rsel):
                hval = plsc.load_gather(rows, [rsel, colidx])
                wrep = plsc.load_gather(wv, [rsel])
                plsc.store_scatter(rows, [rsel, colidx], hval * wrep)
                return rsel + 2

            lax.fori_loop(0, CHUNK * D // 16, mul_body, iota >> 3)
            pltpu.sync_copy(rows, aggsp.at[idx_d], add=True)

        return carry

    lax.fori_loop(0, CPW, body, 0)
    plsc.subcore_barrier()
    pltpu.sync_copy(aggsp.at[pl.ds(s * seg, seg)],
                    agg_hbm.at[c, pl.ds(s * seg, seg)])


_spmv = pl.kernel(
    _spmv_body,
    out_type=jax.ShapeDtypeStruct((NC, NPAD, D), jnp.float32),
    mesh=_mesh,
    compiler_params=pltpu.CompilerParams(needs_layout_passes=False, use_tc_tiling_on_sc=False),
    scratch_types=[
        pltpu.VMEM((CHUNK,), jnp.int32),
        pltpu.VMEM((CHUNK,), jnp.int32),
        pltpu.VMEM((CHUNK,), jnp.float32),
        pltpu.VMEM((CHUNK, D), jnp.float32),
        pltpu.VMEM_SHARED((NPAD, D), jnp.float32),
        pltpu.SemaphoreType.DMA,
    ],
)


def _readout_body(g_hbm, batch_hbm, zeros_hbm, out_hbm, gbuf, bidx, accsp, sem):
    c = lax.axis_index("c")
    s = lax.axis_index("s")
    wid = s * NC + c
    seg = BPAD // NS
    pltpu.sync_copy(zeros_hbm.at[pl.ds(s * seg, seg)],
                    accsp.at[pl.ds(s * seg, seg)])
    plsc.subcore_barrier()

    def body(i, carry):
        g = wid + NW * i

        @pl.when(g < NCHN)
        def _():
            pltpu.sync_copy(g_hbm.at[pl.ds(g * CHUNK, CHUNK)], gbuf)
            pltpu.sync_copy(batch_hbm.at[g], bidx)
            pltpu.sync_copy(gbuf, accsp.at[bidx], add=True)

        return carry

    lax.fori_loop(0, -(-NCHN // NW), body, 0)
    plsc.subcore_barrier()
    pltpu.sync_copy(accsp.at[pl.ds(s * seg, seg)],
                    out_hbm.at[c, pl.ds(s * seg, seg)])


_readout = pl.kernel(
    _readout_body,
    out_type=jax.ShapeDtypeStruct((NC, BPAD), jnp.float32),
    mesh=_mesh,
    compiler_params=pltpu.CompilerParams(needs_layout_passes=False, use_tc_tiling_on_sc=False),
    scratch_types=[
        pltpu.VMEM((CHUNK,), jnp.float32),
        pltpu.VMEM((CHUNK,), jnp.int32),
        pltpu.VMEM_SHARED((BPAD,), jnp.float32),
        pltpu.SemaphoreType.DMA,
    ],
)


def _h1_tc_body(zf_ref, a0_ref, a1_ref, w1_ref, w2_ref, h1_ref):
    zf = zf_ref[...]
    cols = lax.broadcasted_iota(jnp.int32, (CHUNK, D), 1)
    h0 = jnp.where(cols == zf, 1.0, 0.0)
    agg = a0_ref[...] + a1_ref[...]
    t = jnp.dot(jnp.dot(agg, w1_ref[...], preferred_element_type=jnp.float32),
                w2_ref[...], preferred_element_type=jnp.float32)
    h1_ref[...] = h0 + t * jax.nn.sigmoid(t)


def _h2_tc_body(h1_ref, a0_ref, a1_ref, w1_ref, w2_ref, wl_ref, bl_ref, g_ref):
    i = pl.program_id(0)
    agg = a0_ref[...] + a1_ref[...]
    t = jnp.dot(jnp.dot(agg, w1_ref[...], preferred_element_type=jnp.float32),
                w2_ref[...], preferred_element_type=jnp.float32)
    h2 = h1_ref[...] + t * jax.nn.sigmoid(t)
    g = jnp.dot(h2, wl_ref[...], preferred_element_type=jnp.float32) + bl_ref[0, 0]
    rows = i * CHUNK + lax.broadcasted_iota(jnp.int32, (CHUNK, 1), 0)
    g_ref[...] = jnp.where(rows < N, g, 0.0)


def kernel(pos, z, batch, edge_index, W1_0, W2_0, W1_1, W2_1, W_last, b_last):
    f32 = jnp.float32
    src2d = edge_index[0].astype(jnp.int32).reshape(NCH, CHUNK)
    dst2d = edge_index[1].astype(jnp.int32).reshape(NCH, CHUNK)
    pos4 = jnp.zeros((NPAD, 8), f32).at[:N, :3].set(pos)
    zpad = jnp.full((NPAD,), -1, jnp.int32).at[:N].set(z.astype(jnp.int32))
    batch2d = jnp.zeros((NPAD,), jnp.int32).at[:N].set(
        batch.astype(jnp.int32)).reshape(NCHN, CHUNK)
    zeros_flat = jnp.zeros((AGGW,), f32)
    zeros_b = jnp.zeros((BPAD,), f32)

    w2d, aggA = _edge_hist(src2d, dst2d, pos4, zpad, zeros_flat)
    agg1 = aggA.reshape(NC, NPAD, D)

    grid = (NCHN,)
    zf = zpad.reshape(NPAD, 1)
    row_spec = pl.BlockSpec((CHUNK, D), lambda i: (i, 0))
    one_spec = pl.BlockSpec((CHUNK, 1), lambda i: (i, 0))
    w_spec = pl.BlockSpec((D, D), lambda i: (0, 0))
    h1 = pl.pallas_call(
        _h1_tc_body,
        grid=grid,
        in_specs=[one_spec, row_spec, row_spec, w_spec, w_spec],
        out_specs=row_spec,
        out_shape=jax.ShapeDtypeStruct((NPAD, D), f32),
    )(zf, agg1[0], agg1[1], W1_0, W2_0)

    agg2 = _spmv(src2d, dst2d, w2d, h1, zeros_flat.reshape(NPAD, D))

    g = pl.pallas_call(
        _h2_tc_body,
        grid=grid,
        in_specs=[row_spec, row_spec, row_spec, w_spec, w_spec,
                  pl.BlockSpec((D, 1), lambda i: (0, 0)),
                  pl.BlockSpec((1, 1), lambda i: (0, 0))],
        out_specs=one_spec,
        out_shape=jax.ShapeDtypeStruct((NPAD, 1), f32),
    )(h1, agg2[0], agg2[1], W1_1, W2_1, W_last, b_last.reshape(1, 1))

    parts = _readout(g.reshape(NPAD), batch2d, zeros_b)
    return (parts[0] + parts[1])[:B, None]


# speedup vs baseline: 30.0165x; 3.0020x over previous
import jax
import jax.numpy as jnp
from jax import lax
from jax.experimental import pallas as pl
from jax.experimental.pallas import tpu as pltpu
from jax.experimental.pallas import tpu_sc as plsc

N = 100000
E = 3200000
B = 2000
D = 8

CHUNK = 128
KB = 8
NC, NS = 2, 16
NW = NC * NS
CPW = 784
NBLK = CPW // KB
NCHP = NW * CPW
EPAD = NCHP * CHUNK
NPAD = 782 * 128
NCHN = NPAD // CHUNK
AGGW = NPAD * D
BPAD = 2048

_mesh = plsc.VectorSubcoreMesh(core_axis_name="c", subcore_axis_name="s")
_params = pltpu.CompilerParams(needs_layout_passes=False,
                               use_tc_tiling_on_sc=False)


def _edge_hist_body(src_hbm, dst_hbm, pos_hbm, zeros_hbm,
                    w_hbm, agg_hbm,
                    idxS, idxD, prowS, prowD, wblk, flatblk, aggsp, sem):
    c = lax.axis_index("c")
    s = lax.axis_index("s")
    wid = s * NC + c
    seg = AGGW // NS
    pltpu.sync_copy(zeros_hbm.at[pl.ds(s * seg, seg)],
                    aggsp.at[pl.ds(s * seg, seg)])
    plsc.subcore_barrier()

    iota = lax.iota(jnp.int32, 16)
    js0 = jnp.full((16,), 0, jnp.int32)
    js1 = jnp.full((16,), 1, jnp.int32)
    js2 = jnp.full((16,), 2, jnp.int32)
    js3 = jnp.full((16,), 3, jnp.int32)

    def body(b, carry):
        base = wid * CPW + b * KB
        pltpu.sync_copy(src_hbm.at[pl.ds(base, KB)], idxS)
        pltpu.sync_copy(dst_hbm.at[pl.ds(base, KB)], idxD)
        cps = []
        for k in range(KB):
            cps.append(pltpu.async_copy(pos_hbm.at[idxS.at[k]], prowS.at[k], sem))
            cps.append(pltpu.async_copy(pos_hbm.at[idxD.at[k]], prowD.at[k], sem))
        for cp in cps:
            cp.wait()
        for k in range(KB):
            ps_ref = prowS.at[k]
            pd_ref = prowD.at[k]
            for t in range(CHUNK // 16):
                dst16 = idxD[k, pl.ds(t * 16, 16)]
                e16 = t * 16 + iota
                d0 = plsc.load_gather(pd_ref, [e16, js0]) - plsc.load_gather(ps_ref, [e16, js0])
                d1 = plsc.load_gather(pd_ref, [e16, js1]) - plsc.load_gather(ps_ref, [e16, js1])
                d2 = plsc.load_gather(pd_ref, [e16, js2]) - plsc.load_gather(ps_ref, [e16, js2])
                acc = d0 * d0 + d1 * d1 + d2 * d2 + 1e-12
                ib = plsc.bitcast(acc, jnp.int32)
                r = plsc.bitcast(jnp.int32(0x5F3759DF) - (ib >> 1), jnp.float32)
                r = r * (1.5 - 0.5 * acc * r * r)
                r = r * (1.5 - 0.5 * acc * r * r)
                r = r * (1.5 - 0.5 * acc * r * r)
                w16 = jnp.exp(-(acc * r))
                zs = plsc.load_gather(ps_ref, [e16, js3]).astype(jnp.int32)
                wblk[k, pl.ds(t * 16, 16)] = w16
                flatblk[k, pl.ds(t * 16, 16)] = dst16 * D + zs
        pltpu.sync_copy(wblk, w_hbm.at[pl.ds(base, KB)])
        scs = [pltpu.async_copy(wblk.at[k], aggsp.at[flatblk.at[k]], sem, add=True)
               for k in range(KB)]
        for cp in scs:
            cp.wait()
        return carry

    lax.fori_loop(0, NBLK, body, 0)
    plsc.subcore_barrier()
    pltpu.sync_copy(aggsp.at[pl.ds(s * seg, seg)],
                    agg_hbm.at[c, pl.ds(s * seg, seg)])


_edge_hist = pl.kernel(
    _edge_hist_body,
    out_type=(jax.ShapeDtypeStruct((NCHP, CHUNK), jnp.float32),
              jax.ShapeDtypeStruct((NC, AGGW), jnp.float32)),
    mesh=_mesh,
    compiler_params=_params,
    scratch_types=[
        pltpu.VMEM((KB, CHUNK), jnp.int32),
        pltpu.VMEM((KB, CHUNK), jnp.int32),
        pltpu.VMEM((KB, CHUNK, 8), jnp.float32),
        pltpu.VMEM((KB, CHUNK, 8), jnp.float32),
        pltpu.VMEM((KB, CHUNK), jnp.float32),
        pltpu.VMEM((KB, CHUNK), jnp.int32),
        pltpu.VMEM_SHARED((AGGW,), jnp.float32),
        pltpu.SemaphoreType.DMA,
    ],
)


def _spmv_body(src_hbm, dst_hbm, w_hbm, h_hbm, zeros_hbm, agg_hbm,
               idxS, idxD, wblk, rows, aggsp, sem):
    c = lax.axis_index("c")
    s = lax.axis_index("s")
    wid = s * NC + c
    seg = NPAD // NS
    pltpu.sync_copy(zeros_hbm.at[pl.ds(s * seg, seg)],
                    aggsp.at[pl.ds(s * seg, seg)])
    plsc.subcore_barrier()

    iota = lax.iota(jnp.int32, 16)
    colidx = iota & (D - 1)
    rsel0 = iota >> 3

    def body(b, carry):
        base = wid * CPW + b * KB
        pltpu.sync_copy(src_hbm.at[pl.ds(base, KB)], idxS)
        pltpu.sync_copy(dst_hbm.at[pl.ds(base, KB)], idxD)
        pltpu.sync_copy(w_hbm.at[pl.ds(base, KB)], wblk)
        cps = [pltpu.async_copy(h_hbm.at[idxS.at[k]], rows.at[k], sem)
               for k in range(KB)]
        for cp in cps:
            cp.wait()
        for k in range(KB):
            rk = rows.at[k]
            wk = wblk.at[k]

            def mul_body(v, rsel, rk=rk, wk=wk):
                hval = plsc.load_gather(rk, [rsel, colidx])
                wrep = plsc.load_gather(wk, [rsel])
                plsc.store_scatter(rk, [rsel, colidx], hval * wrep)
                return rsel + 2

            lax.fori_loop(0, CHUNK * D // 16, mul_body, rsel0)
        scs = [pltpu.async_copy(rows.at[k], aggsp.at[idxD.at[k]], sem, add=True)
               for k in range(KB)]
        for cp in scs:
            cp.wait()
        return carry

    lax.fori_loop(0, NBLK, body, 0)
    plsc.subcore_barrier()
    pltpu.sync_copy(aggsp.at[pl.ds(s * seg, seg)],
                    agg_hbm.at[c, pl.ds(s * seg, seg)])


_spmv = pl.kernel(
    _spmv_body,
    out_type=jax.ShapeDtypeStruct((NC, NPAD, D), jnp.float32),
    mesh=_mesh,
    compiler_params=_params,
    scratch_types=[
        pltpu.VMEM((KB, CHUNK), jnp.int32),
        pltpu.VMEM((KB, CHUNK), jnp.int32),
        pltpu.VMEM((KB, CHUNK), jnp.float32),
        pltpu.VMEM((KB, CHUNK, D), jnp.float32),
        pltpu.VMEM_SHARED((NPAD, D), jnp.float32),
        pltpu.SemaphoreType.DMA,
    ],
)


def _readout_body(g_hbm, batch_hbm, zeros_hbm, out_hbm, gbuf, bidx, accsp, sem):
    c = lax.axis_index("c")
    s = lax.axis_index("s")
    wid = s * NC + c
    seg = BPAD // NS
    pltpu.sync_copy(zeros_hbm.at[pl.ds(s * seg, seg)],
                    accsp.at[pl.ds(s * seg, seg)])
    plsc.subcore_barrier()

    def body(i, carry):
        g = wid + NW * i

        @pl.when(g < NCHN)
        def _():
            pltpu.sync_copy(g_hbm.at[pl.ds(g * CHUNK, CHUNK)], gbuf)
            pltpu.sync_copy(batch_hbm.at[g], bidx)
            pltpu.sync_copy(gbuf, accsp.at[bidx], add=True)

        return carry

    lax.fori_loop(0, -(-NCHN // NW), body, 0)
    plsc.subcore_barrier()
    pltpu.sync_copy(accsp.at[pl.ds(s * seg, seg)],
                    out_hbm.at[c, pl.ds(s * seg, seg)])


_readout = pl.kernel(
    _readout_body,
    out_type=jax.ShapeDtypeStruct((NC, BPAD), jnp.float32),
    mesh=_mesh,
    compiler_params=_params,
    scratch_types=[
        pltpu.VMEM((CHUNK,), jnp.float32),
        pltpu.VMEM((CHUNK,), jnp.int32),
        pltpu.VMEM_SHARED((BPAD,), jnp.float32),
        pltpu.SemaphoreType.DMA,
    ],
)


S2 = AGGW // 128


def _h1_tc_body(zr_ref, a0_ref, a1_ref, w1_ref, w2_ref, h1_ref):
    cols = lax.broadcasted_iota(jnp.int32, (S2, 128), 1) & (D - 1)
    h0 = jnp.where(cols == zr_ref[...], 1.0, 0.0)
    agg = a0_ref[...] + a1_ref[...]
    t = jnp.dot(jnp.dot(agg, w1_ref[...], preferred_element_type=jnp.float32),
                w2_ref[...], preferred_element_type=jnp.float32)
    h1_ref[...] = h0 + t * jax.nn.sigmoid(t)


def _h2_tc_body(h1_ref, a0_ref, a1_ref, w1_ref, w2_ref, wl_ref, bl_ref, g_ref):
    agg = a0_ref[...] + a1_ref[...]
    t = jnp.dot(jnp.dot(agg, w1_ref[...], preferred_element_type=jnp.float32),
                w2_ref[...], preferred_element_type=jnp.float32)
    h2 = h1_ref[...] + t * jax.nn.sigmoid(t)
    g = jnp.dot(h2, wl_ref[...], preferred_element_type=jnp.float32) + bl_ref[0, 0]
    n = (lax.broadcasted_iota(jnp.int32, (S2, 16), 0) * 16
         + lax.broadcasted_iota(jnp.int32, (S2, 16), 1))
    g_ref[...] = jnp.where(n < N, g, 0.0)


def kernel(pos, z, batch, edge_index, W1_0, W2_0, W1_1, W2_1, W_last, b_last):
    f32 = jnp.float32
    i32 = jnp.int32
    src_p = jnp.full((EPAD,), N, i32).at[:E].set(edge_index[0].astype(i32))
    dst_p = jnp.full((EPAD,), N, i32).at[:E].set(edge_index[1].astype(i32))
    src2d = src_p.reshape(NCHP, CHUNK)
    dst2d = dst_p.reshape(NCHP, CHUNK)
    pos8 = (jnp.zeros((NPAD, 8), f32).at[:N, :3].set(pos)
            .at[:N, 3].set(z.astype(f32)))
    zpad = jnp.full((NPAD,), 5, i32).at[:N].set(z.astype(i32))
    batch2d = jnp.zeros((NPAD,), i32).at[:N].set(
        batch.astype(i32)).reshape(NCHN, CHUNK)
    zeros_flat = jnp.zeros((AGGW,), f32)
    zeros_b = jnp.zeros((BPAD,), f32)

    eye16 = jnp.eye(16, dtype=f32)
    WB1_0 = jnp.kron(eye16, W1_0)
    WB2_0 = jnp.kron(eye16, W2_0)
    WB1_1 = jnp.kron(eye16, W1_1)
    WB2_1 = jnp.kron(eye16, W2_1)
    WLB = jnp.kron(eye16, W_last)
    zrep2d = jnp.repeat(zpad, D).reshape(S2, 128)

    w2d, aggA = _edge_hist(src2d, dst2d, pos8, zeros_flat)
    agg1 = aggA.reshape(NC, S2, 128)

    h1_2d = pl.pallas_call(
        _h1_tc_body,
        out_shape=jax.ShapeDtypeStruct((S2, 128), f32),
    )(zrep2d, agg1[0], agg1[1], WB1_0, WB2_0)

    agg2 = _spmv(src2d, dst2d, w2d, h1_2d.reshape(NPAD, D),
                 zeros_flat.reshape(NPAD, D))
    agg2_2d = agg2.reshape(NC, S2, 128)

    g2d = pl.pallas_call(
        _h2_tc_body,
        out_shape=jax.ShapeDtypeStruct((S2, 16), f32),
    )(h1_2d, agg2_2d[0], agg2_2d[1], WB1_1, WB2_1, WLB, b_last.reshape(1, 1))

    parts = _readout(g2d.reshape(NPAD), batch2d, zeros_b)
    return (parts[0] + parts[1])[:B, None]
